# baseline (speedup 1.0000x reference)
import jax
import jax.numpy as jnp
from jax.experimental import pallas as pl
from jax.experimental.pallas import tpu as pltpu

_B, _S, _D = 4, 2048, 2048
_MAXLEN = 512
_NK, _KB = 8, 256
_PB = _MAXLEN // _NK
_NG = _NK


def _scratch_kernel(pinfo, x1_ref, x2_ref, x3_ref, x4_ref, wa_ref, wb_ref,
                    emb_ref, b_ref, out_ref, mean_ref, acc_ref):
    g = pl.program_id(0)
    pos = pinfo[0]

    @pl.when(g == 0)
    def _():
        acc_ref[...] = jnp.broadcast_to(b_ref[...][None, :], acc_ref.shape)

    ms = jnp.concatenate(
        [jnp.sum(x1_ref[...], axis=1), jnp.sum(x2_ref[...], axis=1),
         jnp.sum(x3_ref[...], axis=1), jnp.sum(x4_ref[...], axis=1)],
        axis=0) * (1.0 / _S)
    mean_ref[:, pl.ds(g * _KB, _KB)] = ms
    ev = emb_ref[0, :, :]
    acc_ref[...] += jax.lax.dot_general(
        ms, wa_ref[...], (((1,), (1,)), ((), ())),
        preferred_element_type=jnp.float32)
    acc_ref[...] += jax.lax.dot_general(
        ev, wb_ref[...], (((1,), (1,)), ((), ())),
        preferred_element_type=jnp.float32)

    out_ref[...] = jnp.zeros_like(out_ref)

    @pl.when(g == _NG - 1)
    def _():
        gate = jax.nn.sigmoid(acc_ref[...])
        val = gate * mean_ref[...]
        out_ref[:, pl.ds(pos % _PB, 1), :] = val[:, None, :]


def _x1_map(g, pinfo):
    return (0, 0, g)


def _x2_map(g, pinfo):
    return (1, 0, g)


def _x3_map(g, pinfo):
    return (2, 0, g)


def _x4_map(g, pinfo):
    return (3, 0, g)


def _wa_map(g, pinfo):
    return (0, g)


def _wb_map(g, pinfo):
    return (0, _NK + g)


def _emb_map(g, pinfo):
    return (pinfo[0], 0, g)


def _b_map(g, pinfo):
    return (0,)


def _out_map(g, pinfo):
    pb = pinfo[0] // _PB
    return (0, (pb + 1 + g) % _NK, 0)


_GRID_SPEC = pltpu.PrefetchScalarGridSpec(
    num_scalar_prefetch=1,
    grid=(_NG,),
    in_specs=[
        pl.BlockSpec((1, _S, _KB), _x1_map),
        pl.BlockSpec((1, _S, _KB), _x2_map),
        pl.BlockSpec((1, _S, _KB), _x3_map),
        pl.BlockSpec((1, _S, _KB), _x4_map),
        pl.BlockSpec((_D, _KB), _wa_map),
        pl.BlockSpec((_D, _KB), _wb_map),
        pl.BlockSpec((1, 1, _KB), _emb_map),
        pl.BlockSpec((_D,), _b_map),
    ],
    out_specs=pl.BlockSpec((_B, _PB, _D), _out_map),
    scratch_shapes=[pltpu.VMEM((_B, _D), jnp.float32),
                    pltpu.VMEM((_B, _D), jnp.float32)],
)


@jax.jit
def _run(current_state, emb_table, W, b, pos):
    pinfo = jnp.reshape(pos, (1,))
    return pl.pallas_call(
        _scratch_kernel,
        grid_spec=_GRID_SPEC,
        out_shape=jax.ShapeDtypeStruct((_B, _MAXLEN, _D), jnp.float32),
        compiler_params=pltpu.CompilerParams(
            dimension_semantics=("arbitrary",)),
    )(pinfo, current_state, current_state, current_state, current_state,
      W, W, emb_table.reshape(_MAXLEN, 1, _D), b)


def kernel(current_state, emb_table, W, b, step):
    pos = jnp.asarray(step, jnp.int32) % _MAXLEN
    return _run(current_state, emb_table, W, b, pos)
